# Initial kernel scaffold; baseline (speedup 1.0000x reference)
import jax
import jax.numpy as jnp
from jax.experimental import pallas as pl


def kernel(x, y, edge_index, edge_type, basis, comp, root, bias, W1, b1, W2, b2):
    raise NotImplementedError("write your pallas kernel here")


# speedup vs baseline: 2.8969x; 2.8969x over previous
import functools

import jax
import jax.numpy as jnp
from jax import lax
from jax.experimental import pallas as pl
from jax.experimental.pallas import tpu as pltpu
from jax.experimental.pallas import tpu_sc as plsc

N = 10000
E = 320000
D = 200
L = 3
R = 2
H = D // 2
W = 128
NC, NS = 2, 16
RPT = 632
NPAD = NS * RPT
NROW = R * N
RPTC = 1256
NROWC = NS * RPTC
CH = 80
EPT = E // NS
NCH = EPT // CH
EPW = E // (NC * NS)
NCH_CNT = EPW // CH
BB = 1000
NG = N // BB


def _seg_body(h2, srcs, comb0, comb1, zrows, a_out,
              a_sh, src_v, comb_v, rows_v, sem):
    c = lax.axis_index("c")
    s = lax.axis_index("s")
    for r, comb in ((0, comb0), (1, comb1)):
        pltpu.sync_copy(zrows, a_sh.at[pl.ds(s * RPT, RPT)])
        plsc.subcore_barrier()

        def chunk(k, carry):
            e0 = s * EPT + k * CH
            pltpu.sync_copy(srcs.at[pl.ds(c * E + e0, CH)], src_v)
            pltpu.sync_copy(comb.at[pl.ds(e0, CH)], comb_v)
            pltpu.async_copy(h2.at[src_v], rows_v, sem).wait()
            pltpu.sync_copy(rows_v, a_sh.at[comb_v], add=True)
            return carry

        lax.fori_loop(0, NCH, chunk, 0)
        plsc.subcore_barrier()
        pltpu.sync_copy(a_sh.at[pl.ds(s * RPT, RPT)],
                        a_out.at[c, r, pl.ds(s * RPT, RPT)])
        plsc.subcore_barrier()


_seg_kernel = pl.kernel(
    _seg_body,
    out_type=jax.ShapeDtypeStruct((NC, R, NPAD, W), jnp.float32),
    mesh=plsc.VectorSubcoreMesh(core_axis_name="c", subcore_axis_name="s"),
    scratch_types=[
        pltpu.VMEM_SHARED((NPAD, W), jnp.float32),
        pltpu.VMEM((CH,), jnp.int32),
        pltpu.VMEM((CH,), jnp.int32),
        pltpu.VMEM((CH, W), jnp.float32),
        pltpu.SemaphoreType.DMA,
    ],
)


def _layer_body(hh_ref, a00_ref, a01_ref, a10_ref, a11_ref,
                basis_ref, comp_ref, root_ref, bias_ref, out_ref, *, leaky):
    h = jnp.concatenate([hh_ref[0][:, :H], hh_ref[1][:, :H]], axis=-1)
    comp = comp_ref[...]
    basis = basis_ref[...]
    w0 = comp[0, 0] * basis[0] + comp[0, 1] * basis[1]
    w1 = comp[1, 0] * basis[0] + comp[1, 1] * basis[1]
    inv0 = 1.0 / jnp.maximum(a00_ref[0, 0][:, H], 1.0)
    inv1 = 1.0 / jnp.maximum(a01_ref[0, 0][:, H], 1.0)
    a0 = jnp.concatenate([a00_ref[0, 0][:, :H], a10_ref[0, 0][:, :H]],
                         axis=-1) * inv0[:, None]
    a1 = jnp.concatenate([a01_ref[0, 0][:, :H], a11_ref[0, 0][:, :H]],
                         axis=-1) * inv1[:, None]
    acc = jnp.dot(a0, w0, preferred_element_type=jnp.float32)
    acc = acc + jnp.dot(a1, w1, preferred_element_type=jnp.float32)
    acc = acc + jnp.dot(h, root_ref[...], preferred_element_type=jnp.float32)
    acc = acc + bias_ref[...]
    if leaky:
        acc = jnp.where(acc > 0, acc, 0.01 * acc)
    one = jnp.ones((BB, 1), jnp.float32)
    pz = jnp.zeros((BB, W - H - 1), jnp.float32)
    out_ref[0] = jnp.concatenate([acc[:, :H], one, pz], axis=-1)
    out_ref[1] = jnp.concatenate([acc[:, H:], jnp.zeros((BB, W - H),
                                                        jnp.float32)], axis=-1)


def _make_layer(leaky):
    return pl.pallas_call(
        functools.partial(_layer_body, leaky=leaky),
        grid=(NG,),
        in_specs=[
            pl.BlockSpec((NC, BB, W), lambda i: (0, i, 0)),
            pl.BlockSpec((1, 1, BB, W), lambda i: (0, 0, i, 0)),
            pl.BlockSpec((1, 1, BB, W), lambda i: (0, 1, i, 0)),
            pl.BlockSpec((1, 1, BB, W), lambda i: (1, 0, i, 0)),
            pl.BlockSpec((1, 1, BB, W), lambda i: (1, 1, i, 0)),
            pl.BlockSpec((2, D, D), lambda i: (0, 0, 0)),
            pl.BlockSpec((R, 2), lambda i: (0, 0)),
            pl.BlockSpec((D, D), lambda i: (0, 0)),
            pl.BlockSpec((1, D), lambda i: (0, 0)),
        ],
        out_specs=pl.BlockSpec((NC, BB, W), lambda i: (0, i, 0)),
        out_shape=jax.ShapeDtypeStruct((NC, N, W), jnp.float32),
    )


_layer_leaky = _make_layer(True)
_layer_plain = _make_layer(False)


def _head_body(y_ref, hrow_ref, w1_ref, b1_ref, w2_ref, b2_ref, out_ref):
    y = y_ref[...]
    h = hrow_ref[...]
    w1 = w1_ref[...]
    d = jnp.abs(y - h)
    p = y * h
    hid = jnp.dot(d, w1[:D], preferred_element_type=jnp.float32)
    hid = hid + jnp.dot(p, w1[D:], preferred_element_type=jnp.float32)
    hid = hid + b1_ref[...]
    hid = jnp.where(hid > 0, hid, 0.01 * hid)
    out = jnp.dot(hid, w2_ref[...], preferred_element_type=jnp.float32)
    out_ref[...] = out + b2_ref[...]


_head_kernel = pl.pallas_call(
    _head_body,
    out_shape=jax.ShapeDtypeStruct((1024, 1), jnp.float32),
)


def kernel(x, y, edge_index, edge_type, basis, comp, root, bias, W1, b1, W2, b2):
    src = edge_index[0].astype(jnp.int32)
    dst = edge_index[1].astype(jnp.int32)
    et = edge_type.astype(jnp.int32)
    srcs = jnp.concatenate([src, src + N])
    dump = N + (jnp.arange(E, dtype=jnp.int32) % (NPAD - N))
    comb0 = jnp.where(et == 0, dst, dump)
    comb1 = jnp.where(et == 1, dst, dump)
    zrows = jnp.zeros((RPT, W), jnp.float32)

    onec = jnp.ones((N, 1), jnp.float32)
    padz = jnp.zeros((N, W - H - 1), jnp.float32)
    pad_x = jnp.zeros((N, W - H), jnp.float32)
    hh = jnp.stack([jnp.concatenate([x[:, :H], onec, padz], axis=-1),
                    jnp.concatenate([x[:, H:], pad_x], axis=-1)])
    for l in range(L):
        h2 = hh.reshape(NC * N, W)
        a = _seg_kernel(h2, srcs, comb0, comb1, zrows)
        layer = _layer_leaky if l != L - 1 else _layer_plain
        hh = layer(hh, a, a, a, a, basis[l], comp[l], root[l],
                   bias[l].reshape(1, D))

    h3row = jnp.concatenate([hh[0, 0, :H], hh[1, 0, :H]]).reshape(1, D)
    return _head_kernel(y, h3row, W1, b1.reshape(1, D), W2, b2.reshape(1, 1))


# speedup vs baseline: 4.8720x; 1.6818x over previous
import functools

import jax
import jax.numpy as jnp
from jax import lax
from jax.experimental import pallas as pl
from jax.experimental.pallas import tpu as pltpu
from jax.experimental.pallas import tpu_sc as plsc

N = 10000
E = 320000
D = 200
L = 3
R = 2
H = D // 2
W = 128
NC, NS = 2, 16
RPT = 632
NPAD = NS * RPT
NROW = R * N
RPTC = 1256
NROWC = NS * RPTC
CH = 80
EPT = E // NS
NCH = EPT // CH
EPW = E // (NC * NS)
NCH_CNT = EPW // CH
BB = 1000
NG = N // BB


def _seg_body(h2, srcs, comb0, comb1, zrows, a_out,
              a_sh, sva, cva, rva, svb, cvb, rvb, sema, semb):
    c = lax.axis_index("c")
    s = lax.axis_index("s")
    for r, comb in ((0, comb0), (1, comb1)):
        pltpu.sync_copy(zrows, a_sh.at[pl.ds(s * RPT, RPT)])
        plsc.subcore_barrier()
        base = s * EPT

        def load_idx(k, sv, cv):
            e0 = base + k * CH
            pltpu.sync_copy(srcs.at[pl.ds(c * E + e0, CH)], sv)
            pltpu.sync_copy(comb.at[pl.ds(e0, CH)], cv)

        load_idx(0, sva, cva)
        pltpu.async_copy(h2.at[sva], rva, sema)

        def body(m, carry):
            k0 = 2 * m
            load_idx(k0 + 1, svb, cvb)
            pltpu.async_copy(h2.at[svb], rvb, semb)
            pltpu.make_async_copy(h2.at[sva], rva, sema).wait()
            pltpu.sync_copy(rva, a_sh.at[cva], add=True)

            @pl.when(m < NCH // 2 - 1)
            def _():
                load_idx(k0 + 2, sva, cva)
                pltpu.async_copy(h2.at[sva], rva, sema)

            pltpu.make_async_copy(h2.at[svb], rvb, semb).wait()
            pltpu.sync_copy(rvb, a_sh.at[cvb], add=True)
            return carry

        lax.fori_loop(0, NCH // 2, body, 0)
        plsc.subcore_barrier()
        pltpu.sync_copy(a_sh.at[pl.ds(s * RPT, RPT)],
                        a_out.at[c, r, pl.ds(s * RPT, RPT)])
        plsc.subcore_barrier()


_seg_kernel = pl.kernel(
    _seg_body,
    out_type=jax.ShapeDtypeStruct((NC, R, NPAD, W), jnp.float32),
    mesh=plsc.VectorSubcoreMesh(core_axis_name="c", subcore_axis_name="s"),
    scratch_types=[
        pltpu.VMEM_SHARED((NPAD, W), jnp.float32),
        pltpu.VMEM((CH,), jnp.int32),
        pltpu.VMEM((CH,), jnp.int32),
        pltpu.VMEM((CH, W), jnp.float32),
        pltpu.VMEM((CH,), jnp.int32),
        pltpu.VMEM((CH,), jnp.int32),
        pltpu.VMEM((CH, W), jnp.float32),
        pltpu.SemaphoreType.DMA,
        pltpu.SemaphoreType.DMA,
    ],
)


def _layer_body(hh_ref, a00_ref, a01_ref, a10_ref, a11_ref,
                basis_ref, comp_ref, root_ref, bias_ref, out_ref, *, leaky):
    h = jnp.concatenate([hh_ref[0][:, :H], hh_ref[1][:, :H]], axis=-1)
    comp = comp_ref[...]
    basis = basis_ref[...]
    w0 = comp[0, 0] * basis[0] + comp[0, 1] * basis[1]
    w1 = comp[1, 0] * basis[0] + comp[1, 1] * basis[1]
    inv0 = 1.0 / jnp.maximum(a00_ref[0, 0][:, H], 1.0)
    inv1 = 1.0 / jnp.maximum(a01_ref[0, 0][:, H], 1.0)
    a0 = jnp.concatenate([a00_ref[0, 0][:, :H], a10_ref[0, 0][:, :H]],
                         axis=-1) * inv0[:, None]
    a1 = jnp.concatenate([a01_ref[0, 0][:, :H], a11_ref[0, 0][:, :H]],
                         axis=-1) * inv1[:, None]
    acc = jnp.dot(a0, w0, preferred_element_type=jnp.float32)
    acc = acc + jnp.dot(a1, w1, preferred_element_type=jnp.float32)
    acc = acc + jnp.dot(h, root_ref[...], preferred_element_type=jnp.float32)
    acc = acc + bias_ref[...]
    if leaky:
        acc = jnp.where(acc > 0, acc, 0.01 * acc)
    one = jnp.ones((BB, 1), jnp.float32)
    pz = jnp.zeros((BB, W - H - 1), jnp.float32)
    out_ref[0] = jnp.concatenate([acc[:, :H], one, pz], axis=-1)
    out_ref[1] = jnp.concatenate([acc[:, H:], jnp.zeros((BB, W - H),
                                                        jnp.float32)], axis=-1)


def _make_layer(leaky):
    return pl.pallas_call(
        functools.partial(_layer_body, leaky=leaky),
        grid=(NG,),
        in_specs=[
            pl.BlockSpec((NC, BB, W), lambda i: (0, i, 0)),
            pl.BlockSpec((1, 1, BB, W), lambda i: (0, 0, i, 0)),
            pl.BlockSpec((1, 1, BB, W), lambda i: (0, 1, i, 0)),
            pl.BlockSpec((1, 1, BB, W), lambda i: (1, 0, i, 0)),
            pl.BlockSpec((1, 1, BB, W), lambda i: (1, 1, i, 0)),
            pl.BlockSpec((2, D, D), lambda i: (0, 0, 0)),
            pl.BlockSpec((R, 2), lambda i: (0, 0)),
            pl.BlockSpec((D, D), lambda i: (0, 0)),
            pl.BlockSpec((1, D), lambda i: (0, 0)),
        ],
        out_specs=pl.BlockSpec((NC, BB, W), lambda i: (0, i, 0)),
        out_shape=jax.ShapeDtypeStruct((NC, N, W), jnp.float32),
    )


_layer_leaky = _make_layer(True)
_layer_plain = _make_layer(False)


def _head_body(y_ref, hrow_ref, w1_ref, b1_ref, w2_ref, b2_ref, out_ref):
    y = y_ref[...]
    h = hrow_ref[...]
    w1 = w1_ref[...]
    d = jnp.abs(y - h)
    p = y * h
    hid = jnp.dot(d, w1[:D], preferred_element_type=jnp.float32)
    hid = hid + jnp.dot(p, w1[D:], preferred_element_type=jnp.float32)
    hid = hid + b1_ref[...]
    hid = jnp.where(hid > 0, hid, 0.01 * hid)
    out = jnp.dot(hid, w2_ref[...], preferred_element_type=jnp.float32)
    out_ref[...] = out + b2_ref[...]


_head_kernel = pl.pallas_call(
    _head_body,
    out_shape=jax.ShapeDtypeStruct((1024, 1), jnp.float32),
)


def kernel(x, y, edge_index, edge_type, basis, comp, root, bias, W1, b1, W2, b2):
    src = edge_index[0].astype(jnp.int32)
    dst = edge_index[1].astype(jnp.int32)
    et = edge_type.astype(jnp.int32)
    srcs = jnp.concatenate([src, src + N])
    dump = N + (jnp.arange(E, dtype=jnp.int32) % (NPAD - N))
    comb0 = jnp.where(et == 0, dst, dump)
    comb1 = jnp.where(et == 1, dst, dump)
    zrows = jnp.zeros((RPT, W), jnp.float32)

    onec = jnp.ones((N, 1), jnp.float32)
    padz = jnp.zeros((N, W - H - 1), jnp.float32)
    pad_x = jnp.zeros((N, W - H), jnp.float32)
    hh = jnp.stack([jnp.concatenate([x[:, :H], onec, padz], axis=-1),
                    jnp.concatenate([x[:, H:], pad_x], axis=-1)])
    for l in range(L):
        h2 = hh.reshape(NC * N, W)
        a = _seg_kernel(h2, srcs, comb0, comb1, zrows)
        layer = _layer_leaky if l != L - 1 else _layer_plain
        hh = layer(hh, a, a, a, a, basis[l], comp[l], root[l],
                   bias[l].reshape(1, D))

    h3row = jnp.concatenate([hh[0, 0, :H], hh[1, 0, :H]]).reshape(1, D)
    return _head_kernel(y, h3row, W1, b1.reshape(1, D), W2, b2.reshape(1, 1))


# speedup vs baseline: 6.0385x; 1.2394x over previous
import functools

import jax
import jax.numpy as jnp
from jax import lax
from jax.experimental import pallas as pl
from jax.experimental.pallas import tpu as pltpu
from jax.experimental.pallas import tpu_sc as plsc

N = 10000
E = 320000
D = 200
L = 3
R = 2
H = D // 2
W = 128
NC, NS = 2, 16
RPT = 632
NPAD = NS * RPT
NROW = R * N
RPTC = 1256
NROWC = NS * RPTC
CH = 80
EPT = E // NS
NCH = EPT // CH
EPW = E // (NC * NS)
NCH_CNT = EPW // CH
BB = 1000
NG = N // BB


NSLOT = 4
NFULL = (NCH - 2) // NSLOT


def _seg_body(h2, srcs, comb0, comb1, zrows, a_out, a_sh, *bufs):
    sv = bufs[0:4]
    cv = bufs[4:8]
    rv = bufs[8:12]
    gsem = bufs[12:16]
    ssem = bufs[16:20]
    c = lax.axis_index("c")
    s = lax.axis_index("s")
    for r, comb in ((0, comb0), (1, comb1)):
        pltpu.sync_copy(zrows, a_sh.at[pl.ds(s * RPT, RPT)])
        plsc.subcore_barrier()
        base = s * EPT

        def load_idx(k, b):
            e0 = base + k * CH
            pltpu.sync_copy(srcs.at[pl.ds(c * E + e0, CH)], sv[b])
            pltpu.sync_copy(comb.at[pl.ds(e0, CH)], cv[b])

        def fire_gather(b):
            pltpu.async_copy(h2.at[sv[b]], rv[b], gsem[b])

        def wait_gather(b):
            pltpu.make_async_copy(h2.at[sv[b]], rv[b], gsem[b]).wait()

        def fire_scat(b):
            pltpu.async_copy(rv[b], a_sh.at[cv[b]], ssem[b], add=True)

        def wait_scat(b):
            pltpu.make_async_copy(rv[b], a_sh.at[cv[b]], ssem[b]).wait()

        load_idx(0, 0)
        fire_gather(0)
        load_idx(1, 1)
        fire_gather(1)

        def body(m, carry):
            k0 = NSLOT * m
            for b in range(NSLOT):
                wait_gather(b)
                fire_scat(b)
                nb = (b + 2) % NSLOT
                if b < 2:
                    @pl.when(m >= 1)
                    def _():
                        wait_scat(nb)
                else:
                    wait_scat(nb)
                load_idx(k0 + b + 2, nb)
                fire_gather(nb)
            return carry

        lax.fori_loop(0, NFULL, body, 0)
        wait_gather(0)
        fire_scat(0)
        wait_gather(1)
        fire_scat(1)
        for b in range(NSLOT):
            wait_scat(b)
        plsc.subcore_barrier()
        pltpu.sync_copy(a_sh.at[pl.ds(s * RPT, RPT)],
                        a_out.at[c, r, pl.ds(s * RPT, RPT)])
        plsc.subcore_barrier()


_seg_kernel = pl.kernel(
    _seg_body,
    out_type=jax.ShapeDtypeStruct((NC, R, NPAD, W), jnp.float32),
    mesh=plsc.VectorSubcoreMesh(core_axis_name="c", subcore_axis_name="s"),
    scratch_types=[pltpu.VMEM_SHARED((NPAD, W), jnp.float32)]
    + [pltpu.VMEM((CH,), jnp.int32)] * 4
    + [pltpu.VMEM((CH,), jnp.int32)] * 4
    + [pltpu.VMEM((CH, W), jnp.float32)] * 4
    + [pltpu.SemaphoreType.DMA] * 8,
)


def _layer_body(hh_ref, a00_ref, a01_ref, a10_ref, a11_ref,
                basis_ref, comp_ref, root_ref, bias_ref, out_ref, *, leaky):
    h = jnp.concatenate([hh_ref[0][:, :H], hh_ref[1][:, :H]], axis=-1)
    comp = comp_ref[...]
    basis = basis_ref[...]
    w0 = comp[0, 0] * basis[0] + comp[0, 1] * basis[1]
    w1 = comp[1, 0] * basis[0] + comp[1, 1] * basis[1]
    inv0 = 1.0 / jnp.maximum(a00_ref[0, 0][:, H], 1.0)
    inv1 = 1.0 / jnp.maximum(a01_ref[0, 0][:, H], 1.0)
    a0 = jnp.concatenate([a00_ref[0, 0][:, :H], a10_ref[0, 0][:, :H]],
                         axis=-1) * inv0[:, None]
    a1 = jnp.concatenate([a01_ref[0, 0][:, :H], a11_ref[0, 0][:, :H]],
                         axis=-1) * inv1[:, None]
    acc = jnp.dot(a0, w0, preferred_element_type=jnp.float32)
    acc = acc + jnp.dot(a1, w1, preferred_element_type=jnp.float32)
    acc = acc + jnp.dot(h, root_ref[...], preferred_element_type=jnp.float32)
    acc = acc + bias_ref[...]
    if leaky:
        acc = jnp.where(acc > 0, acc, 0.01 * acc)
    one = jnp.ones((BB, 1), jnp.float32)
    pz = jnp.zeros((BB, W - H - 1), jnp.float32)
    out_ref[0] = jnp.concatenate([acc[:, :H], one, pz], axis=-1)
    out_ref[1] = jnp.concatenate([acc[:, H:], jnp.zeros((BB, W - H),
                                                        jnp.float32)], axis=-1)


def _make_layer(leaky):
    return pl.pallas_call(
        functools.partial(_layer_body, leaky=leaky),
        grid=(NG,),
        in_specs=[
            pl.BlockSpec((NC, BB, W), lambda i: (0, i, 0)),
            pl.BlockSpec((1, 1, BB, W), lambda i: (0, 0, i, 0)),
            pl.BlockSpec((1, 1, BB, W), lambda i: (0, 1, i, 0)),
            pl.BlockSpec((1, 1, BB, W), lambda i: (1, 0, i, 0)),
            pl.BlockSpec((1, 1, BB, W), lambda i: (1, 1, i, 0)),
            pl.BlockSpec((2, D, D), lambda i: (0, 0, 0)),
            pl.BlockSpec((R, 2), lambda i: (0, 0)),
            pl.BlockSpec((D, D), lambda i: (0, 0)),
            pl.BlockSpec((1, D), lambda i: (0, 0)),
        ],
        out_specs=pl.BlockSpec((NC, BB, W), lambda i: (0, i, 0)),
        out_shape=jax.ShapeDtypeStruct((NC, N, W), jnp.float32),
    )


_layer_leaky = _make_layer(True)
_layer_plain = _make_layer(False)


def _head_body(y_ref, hrow_ref, w1_ref, b1_ref, w2_ref, b2_ref, out_ref):
    y = y_ref[...]
    h = hrow_ref[...]
    w1 = w1_ref[...]
    d = jnp.abs(y - h)
    p = y * h
    hid = jnp.dot(d, w1[:D], preferred_element_type=jnp.float32)
    hid = hid + jnp.dot(p, w1[D:], preferred_element_type=jnp.float32)
    hid = hid + b1_ref[...]
    hid = jnp.where(hid > 0, hid, 0.01 * hid)
    out = jnp.dot(hid, w2_ref[...], preferred_element_type=jnp.float32)
    out_ref[...] = out + b2_ref[...]


_head_kernel = pl.pallas_call(
    _head_body,
    out_shape=jax.ShapeDtypeStruct((1024, 1), jnp.float32),
)


def kernel(x, y, edge_index, edge_type, basis, comp, root, bias, W1, b1, W2, b2):
    src = edge_index[0].astype(jnp.int32)
    dst = edge_index[1].astype(jnp.int32)
    et = edge_type.astype(jnp.int32)
    srcs = jnp.concatenate([src, src + N])
    dump = N + (jnp.arange(E, dtype=jnp.int32) % (NPAD - N))
    comb0 = jnp.where(et == 0, dst, dump)
    comb1 = jnp.where(et == 1, dst, dump)
    zrows = jnp.zeros((RPT, W), jnp.float32)

    onec = jnp.ones((N, 1), jnp.float32)
    padz = jnp.zeros((N, W - H - 1), jnp.float32)
    pad_x = jnp.zeros((N, W - H), jnp.float32)
    hh = jnp.stack([jnp.concatenate([x[:, :H], onec, padz], axis=-1),
                    jnp.concatenate([x[:, H:], pad_x], axis=-1)])
    for l in range(L):
        h2 = hh.reshape(NC * N, W)
        a = _seg_kernel(h2, srcs, comb0, comb1, zrows)
        layer = _layer_leaky if l != L - 1 else _layer_plain
        hh = layer(hh, a, a, a, a, basis[l], comp[l], root[l],
                   bias[l].reshape(1, D))

    h3row = jnp.concatenate([hh[0, 0, :H], hh[1, 0, :H]]).reshape(1, D)
    return _head_kernel(y, h3row, W1, b1.reshape(1, D), W2, b2.reshape(1, 1))
